# Initial kernel scaffold; baseline (speedup 1.0000x reference)
import jax
import jax.numpy as jnp
from jax.experimental import pallas as pl


def kernel(x, ln_w, ln_b):
    raise NotImplementedError("write your pallas kernel here")


# speedup vs baseline: 16.2517x; 16.2517x over previous
import functools

import jax
import jax.numpy as jnp
from jax.experimental import pallas as pl


def _rowblock_kernel(x_ref, w_ref, b_ref, out_ref, mask_ref, *, k):
    x = x_ref[...]
    n = x.shape[1]
    mean = jnp.mean(x, axis=1, keepdims=True)
    xc = x - mean
    var = jnp.mean(xc * xc, axis=1, keepdims=True)
    xn = xc * jax.lax.rsqrt(var + 1e-5)
    xn = xn * w_ref[...] + b_ref[...]
    a = 0.5 * xn * (1.0 + jax.lax.erf(xn * 0.7071067811865476))

    bits = jax.lax.bitcast_convert_type(a, jnp.uint32)
    neg = bits >= jnp.uint32(0x80000000)
    keys = jnp.where(neg, ~bits, bits | jnp.uint32(0x80000000))

    def body(i, t):
        bit = jnp.left_shift(jnp.uint32(1), (31 - i).astype(jnp.uint32))
        cand = t | bit
        cnt = jnp.sum((keys >= cand).astype(jnp.int32), axis=1, keepdims=True)
        return jnp.where(cnt >= k, cand, t)

    t0 = jnp.zeros((x.shape[0], 1), dtype=jnp.uint32)
    t = jax.lax.fori_loop(0, 32, body, t0)

    mask = (keys >= t).astype(jnp.float32)
    mask_ref[...] = mask
    out_ref[...] = a * mask


def kernel(x, ln_w, ln_b):
    b, n = x.shape
    k = max(1, int(0.1 * n))
    rows = 8 if b % 8 == 0 else 1
    grid = (b // rows,)
    out, mask = pl.pallas_call(
        functools.partial(_rowblock_kernel, k=k),
        grid=grid,
        in_specs=[
            pl.BlockSpec((rows, n), lambda i: (i, 0)),
            pl.BlockSpec((1, n), lambda i: (0, 0)),
            pl.BlockSpec((1, n), lambda i: (0, 0)),
        ],
        out_specs=[
            pl.BlockSpec((rows, n), lambda i: (i, 0)),
            pl.BlockSpec((rows, n), lambda i: (i, 0)),
        ],
        out_shape=[
            jax.ShapeDtypeStruct((b, n), jnp.float32),
            jax.ShapeDtypeStruct((b, n), jnp.float32),
        ],
    )(x, ln_w.reshape(1, n), ln_b.reshape(1, n))
    return (out, mask)


# speedup vs baseline: 39.0289x; 2.4015x over previous
import functools

import jax
import jax.numpy as jnp
from jax.experimental import pallas as pl


def _rowblock_kernel(x_ref, w_ref, b_ref, out_ref, mask_ref, *, k):
    x = x_ref[...]
    rows = x.shape[0]
    mean = jnp.mean(x, axis=1, keepdims=True)
    xc = x - mean
    var = jnp.mean(xc * xc, axis=1, keepdims=True)
    xn = xc * jax.lax.rsqrt(var + 1e-5)
    xn = xn * w_ref[...] + b_ref[...]
    a = 0.5 * xn * (1.0 + jax.lax.erf(xn * 0.7071067811865476))

    bits = jax.lax.bitcast_convert_type(a, jnp.uint32)
    neg = bits >= jnp.uint32(0x80000000)
    keys = jnp.where(neg, ~bits, bits | jnp.uint32(0x80000000))
    hi = ((keys >> jnp.uint32(16)) ^ jnp.uint32(0x8000)).astype(jnp.int16)
    lo = (keys ^ jnp.uint32(0x8000)).astype(jnp.int16)

    neg_one = jnp.int16(-1)
    zero16 = jnp.int16(0)
    neg_k = jnp.full((rows, 1), -k, dtype=jnp.int32)

    def count_neg(hits):
        w = hits.shape[1]
        while w > 128:
            w //= 2
            hits = hits[:, :w] + hits[:, w:]
        return jnp.sum(hits.astype(jnp.int32), axis=1, keepdims=True)

    def select_pass(data16, rank_neg):
        def body(i, t32):
            cand32 = t32 | jnp.left_shift(jnp.int32(1), 15 - i)
            cand16 = (cand32 ^ jnp.int32(0x8000)).astype(jnp.int16)
            cnt = count_neg(jnp.where(data16 >= cand16, neg_one, zero16))
            return jnp.where(cnt <= rank_neg, cand32, t32)

        t0 = jnp.zeros((rows, 1), dtype=jnp.int32)
        return jax.lax.fori_loop(0, 16, body, t0)

    t_hi32 = select_pass(hi, neg_k)
    t_hi16 = (t_hi32 ^ jnp.int32(0x8000)).astype(jnp.int16)

    above = count_neg(jnp.where(hi > t_hi16, neg_one, zero16))
    neg_s = neg_k - above
    lo_m = jnp.where(hi == t_hi16, lo, jnp.int16(-32768))
    t_lo32 = select_pass(lo_m, neg_s)

    thresh = (t_hi32.astype(jnp.uint32) << jnp.uint32(16)) | t_lo32.astype(
        jnp.uint32)
    mask = (keys >= thresh).astype(jnp.float32)
    mask_ref[...] = mask
    out_ref[...] = a * mask


def kernel(x, ln_w, ln_b):
    b, n = x.shape
    k = max(1, int(0.1 * n))
    rows = 16 if b % 16 == 0 else 1
    grid = (b // rows,)
    out, mask = pl.pallas_call(
        functools.partial(_rowblock_kernel, k=k),
        grid=grid,
        in_specs=[
            pl.BlockSpec((rows, n), lambda i: (i, 0)),
            pl.BlockSpec((1, n), lambda i: (0, 0)),
            pl.BlockSpec((1, n), lambda i: (0, 0)),
        ],
        out_specs=[
            pl.BlockSpec((rows, n), lambda i: (i, 0)),
            pl.BlockSpec((rows, n), lambda i: (i, 0)),
        ],
        out_shape=[
            jax.ShapeDtypeStruct((b, n), jnp.float32),
            jax.ShapeDtypeStruct((b, n), jnp.float32),
        ],
    )(x, ln_w.reshape(1, n), ln_b.reshape(1, n))
    return (out, mask)


# speedup vs baseline: 47.7761x; 1.2241x over previous
import functools

import jax
import jax.numpy as jnp
from jax.experimental import pallas as pl


def _rowblock_kernel(x_ref, w_ref, b_ref, out_ref, mask_ref, *, k):
    x = x_ref[...]
    rows = x.shape[0]
    mean = jnp.mean(x, axis=1, keepdims=True)
    xc = x - mean
    var = jnp.mean(xc * xc, axis=1, keepdims=True)
    xn = xc * jax.lax.rsqrt(var + 1e-5)
    xn = xn * w_ref[...] + b_ref[...]
    a = 0.5 * xn * (1.0 + jax.lax.erf(xn * 0.7071067811865476))

    bits_i = jax.lax.bitcast_convert_type(a, jnp.int32)
    flip = (bits_i >> 31) | jnp.int32(-0x80000000)
    keys = jax.lax.bitcast_convert_type(bits_i ^ flip, jnp.uint32)
    hi = ((keys >> jnp.uint32(16)) ^ jnp.uint32(0x8000)).astype(jnp.int16)
    lo = (keys ^ jnp.uint32(0x8000)).astype(jnp.int16)

    neg_one = jnp.int16(-1)
    zero16 = jnp.int16(0)
    neg_k = jnp.full((rows, 1), -k, dtype=jnp.int32)

    def count_neg(hits):
        w = hits.shape[1]
        while w > 128:
            w //= 2
            hits = hits[:, :w] + hits[:, w:]
        return jnp.sum(hits.astype(jnp.int32), axis=1, keepdims=True)

    def select_pass(data16, rank_neg):
        def body(i, t32):
            cand32 = t32 | jnp.left_shift(jnp.int32(1), 15 - i)
            cand16 = (cand32 ^ jnp.int32(0x8000)).astype(jnp.int16)
            cnt = count_neg(jnp.where(data16 >= cand16, neg_one, zero16))
            return jnp.where(cnt <= rank_neg, cand32, t32)

        t0 = jnp.zeros((rows, 1), dtype=jnp.int32)
        return jax.lax.fori_loop(0, 16, body, t0)

    t_hi32 = select_pass(hi, neg_k)
    t_hi16 = (t_hi32 ^ jnp.int32(0x8000)).astype(jnp.int16)

    above = count_neg(jnp.where(hi > t_hi16, neg_one, zero16))
    neg_s = neg_k - above
    lo_m = jnp.where(hi == t_hi16, lo, jnp.int16(-32768))
    t_lo32 = select_pass(lo_m, neg_s)

    thresh = (t_hi32.astype(jnp.uint32) << jnp.uint32(16)) | t_lo32.astype(
        jnp.uint32)
    mask = (keys >= thresh).astype(jnp.float32)
    mask_ref[...] = mask
    out_ref[...] = a * mask


def kernel(x, ln_w, ln_b):
    b, n = x.shape
    k = max(1, int(0.1 * n))
    rows = 32 if b % 32 == 0 else 1
    grid = (b // rows,)
    out, mask = pl.pallas_call(
        functools.partial(_rowblock_kernel, k=k),
        grid=grid,
        in_specs=[
            pl.BlockSpec((rows, n), lambda i: (i, 0)),
            pl.BlockSpec((1, n), lambda i: (0, 0)),
            pl.BlockSpec((1, n), lambda i: (0, 0)),
        ],
        out_specs=[
            pl.BlockSpec((rows, n), lambda i: (i, 0)),
            pl.BlockSpec((rows, n), lambda i: (i, 0)),
        ],
        out_shape=[
            jax.ShapeDtypeStruct((b, n), jnp.float32),
            jax.ShapeDtypeStruct((b, n), jnp.float32),
        ],
    )(x, ln_w.reshape(1, n), ln_b.reshape(1, n))
    return (out, mask)


# speedup vs baseline: 47.8212x; 1.0009x over previous
import functools

import jax
import jax.numpy as jnp
from jax.experimental import pallas as pl


def _rowblock_kernel(x_ref, w_ref, b_ref, out_ref, mask_ref, *, k):
    x = x_ref[...]
    rows = x.shape[0]
    n_inv = 1.0 / x.shape[1]
    s1 = jnp.sum(x, axis=1, keepdims=True)
    s2 = jnp.sum(x * x, axis=1, keepdims=True)
    mean = s1 * n_inv
    var = s2 * n_inv - mean * mean
    xn = (x - mean) * jax.lax.rsqrt(var + 1e-5)
    xn = xn * w_ref[...] + b_ref[...]
    a = 0.5 * xn * (1.0 + jax.lax.erf(xn * 0.7071067811865476))

    bits_i = jax.lax.bitcast_convert_type(a, jnp.int32)
    flip = (bits_i >> 31) | jnp.int32(-0x80000000)
    keys = jax.lax.bitcast_convert_type(bits_i ^ flip, jnp.uint32)
    hi = ((keys >> jnp.uint32(16)) ^ jnp.uint32(0x8000)).astype(jnp.int16)
    lo = (keys ^ jnp.uint32(0x8000)).astype(jnp.int16)

    neg_one = jnp.int16(-1)
    zero16 = jnp.int16(0)
    neg_k = jnp.full((rows, 1), -k, dtype=jnp.int32)

    chunk = 1024

    def fused_count(data16, cand16, strict):
        n = data16.shape[1]
        acc = jnp.zeros((rows, chunk), dtype=jnp.int16)
        for g in range(n // chunk):
            sl = data16[:, g * chunk:(g + 1) * chunk]
            pred = sl > cand16 if strict else sl >= cand16
            acc = acc + jnp.where(pred, neg_one, zero16)
        w = chunk
        while w > 128:
            w //= 2
            acc = acc[:, :w] + acc[:, w:]
        return jnp.sum(acc.astype(jnp.int32), axis=1, keepdims=True)

    def select_pass(data16, rank_neg):
        def body(i, t32):
            cand32 = t32 | jnp.left_shift(jnp.int32(1), 15 - i)
            cand16 = (cand32 ^ jnp.int32(0x8000)).astype(jnp.int16)
            cnt = fused_count(data16, cand16, strict=False)
            return jnp.where(cnt <= rank_neg, cand32, t32)

        t0 = jnp.zeros((rows, 1), dtype=jnp.int32)
        return jax.lax.fori_loop(0, 16, body, t0)

    t_hi32 = select_pass(hi, neg_k)
    t_hi16 = (t_hi32 ^ jnp.int32(0x8000)).astype(jnp.int16)

    above = fused_count(hi, t_hi16, strict=True)
    neg_s = neg_k - above
    lo_m = jnp.where(hi == t_hi16, lo, jnp.int16(-32768))
    t_lo32 = select_pass(lo_m, neg_s)

    thresh = (t_hi32.astype(jnp.uint32) << jnp.uint32(16)) | t_lo32.astype(
        jnp.uint32)
    mask = (keys >= thresh).astype(jnp.float32)
    mask_ref[...] = mask
    out_ref[...] = a * mask


def kernel(x, ln_w, ln_b):
    b, n = x.shape
    k = max(1, int(0.1 * n))
    rows = 32 if b % 32 == 0 else 1
    grid = (b // rows,)
    out, mask = pl.pallas_call(
        functools.partial(_rowblock_kernel, k=k),
        grid=grid,
        in_specs=[
            pl.BlockSpec((rows, n), lambda i: (i, 0)),
            pl.BlockSpec((1, n), lambda i: (0, 0)),
            pl.BlockSpec((1, n), lambda i: (0, 0)),
        ],
        out_specs=[
            pl.BlockSpec((rows, n), lambda i: (i, 0)),
            pl.BlockSpec((rows, n), lambda i: (i, 0)),
        ],
        out_shape=[
            jax.ShapeDtypeStruct((b, n), jnp.float32),
            jax.ShapeDtypeStruct((b, n), jnp.float32),
        ],
    )(x, ln_w.reshape(1, n), ln_b.reshape(1, n))
    return (out, mask)


# speedup vs baseline: 48.6120x; 1.0165x over previous
import functools

import jax
import jax.numpy as jnp
from jax.experimental import pallas as pl


def _rowblock_kernel(x_ref, w_ref, b_ref, out_ref, mask_ref, *, k):
    x = x_ref[...]
    rows = x.shape[0]
    n_inv = 1.0 / x.shape[1]
    s1 = jnp.sum(x, axis=1, keepdims=True)
    s2 = jnp.sum(x * x, axis=1, keepdims=True)
    mean = s1 * n_inv
    var = s2 * n_inv - mean * mean
    xn = (x - mean) * jax.lax.rsqrt(var + 1e-5)
    xn = xn * w_ref[...] + b_ref[...]
    a = xn * (0.5 * jax.lax.erf(xn * 0.7071067811865476) + 0.5)

    bits_i = jax.lax.bitcast_convert_type(a, jnp.int32)
    flip = (bits_i >> 31) | jnp.int32(-0x80000000)
    keys = jax.lax.bitcast_convert_type(bits_i ^ flip, jnp.uint32)
    hi = ((keys >> jnp.uint32(16)) ^ jnp.uint32(0x8000)).astype(jnp.int16)
    lo = (keys ^ jnp.uint32(0x8000)).astype(jnp.int16)

    neg_one = jnp.int16(-1)
    zero16 = jnp.int16(0)
    neg_k = jnp.full((rows, 1), -k, dtype=jnp.int32)

    chunk = 1024

    def fused_count(data16, cand16, strict):
        n = data16.shape[1]
        acc = jnp.zeros((rows, chunk), dtype=jnp.int16)
        for g in range(n // chunk):
            sl = data16[:, g * chunk:(g + 1) * chunk]
            pred = sl > cand16 if strict else sl >= cand16
            acc = acc + jnp.where(pred, neg_one, zero16)
        w = chunk
        while w > 128:
            w //= 2
            acc = acc[:, :w] + acc[:, w:]
        return jnp.sum(acc.astype(jnp.int32), axis=1, keepdims=True)

    def select_pass(data16, rank_neg):
        def body(i, t32):
            cand32 = t32 | jnp.left_shift(jnp.int32(1), 15 - i)
            cand16 = (cand32 ^ jnp.int32(0x8000)).astype(jnp.int16)
            cnt = fused_count(data16, cand16, strict=False)
            return jnp.where(cnt <= rank_neg, cand32, t32)

        t0 = jnp.zeros((rows, 1), dtype=jnp.int32)
        return jax.lax.fori_loop(0, 16, body, t0, unroll=2)

    t_hi32 = select_pass(hi, neg_k)
    t_hi16 = (t_hi32 ^ jnp.int32(0x8000)).astype(jnp.int16)

    above = fused_count(hi, t_hi16, strict=True)
    neg_s = neg_k - above
    lo_m = jnp.where(hi == t_hi16, lo, jnp.int16(-32768))
    t_lo32 = select_pass(lo_m, neg_s)

    thresh = (t_hi32.astype(jnp.uint32) << jnp.uint32(16)) | t_lo32.astype(
        jnp.uint32)
    sel = keys >= thresh
    mask_ref[...] = jnp.where(sel, 1.0, 0.0).astype(jnp.float32)
    out_ref[...] = jnp.where(sel, a, 0.0)


def kernel(x, ln_w, ln_b):
    b, n = x.shape
    k = max(1, int(0.1 * n))
    rows = 32 if b % 32 == 0 else 1
    grid = (b // rows,)
    out, mask = pl.pallas_call(
        functools.partial(_rowblock_kernel, k=k),
        grid=grid,
        in_specs=[
            pl.BlockSpec((rows, n), lambda i: (i, 0)),
            pl.BlockSpec((1, n), lambda i: (0, 0)),
            pl.BlockSpec((1, n), lambda i: (0, 0)),
        ],
        out_specs=[
            pl.BlockSpec((rows, n), lambda i: (i, 0)),
            pl.BlockSpec((rows, n), lambda i: (i, 0)),
        ],
        out_shape=[
            jax.ShapeDtypeStruct((b, n), jnp.float32),
            jax.ShapeDtypeStruct((b, n), jnp.float32),
        ],
    )(x, ln_w.reshape(1, n), ln_b.reshape(1, n))
    return (out, mask)
